# baseline (speedup 1.0000x reference)
import jax
import jax.numpy as jnp
from jax import lax
from jax.experimental import pallas as pl

B, L, V, E, H, C = 16384, 5, 1000000, 64, 128, 2

_BLK = 2048


def _mlp_t_body(x_refs, w1t_ref, b1_ref, w2t_ref, b2_ref, w3t_ref, b3_ref, o_ref):
    w1t = w1t_ref[...]
    acc = jnp.zeros((H, _BLK), jnp.float32)
    for l in range(L):
        acc = acc + jnp.dot(w1t[:, l * E:(l + 1) * E], x_refs[l][...],
                            preferred_element_type=jnp.float32)
    h = jnp.maximum(acc + b1_ref[...], 0.0).astype(jnp.bfloat16)
    h = jnp.maximum(jnp.dot(w2t_ref[...], h,
                            preferred_element_type=jnp.float32) + b2_ref[...], 0.0)
    h = h.astype(jnp.bfloat16)
    o = jnp.dot(w3t_ref[...], h, preferred_element_type=jnp.float32) + b3_ref[...]
    row = lax.broadcasted_iota(jnp.int32, o.shape, 0)
    o = jnp.where(row < C, o, jnp.float32(-1e30))
    m = jnp.max(o, axis=0, keepdims=True)
    e = jnp.exp(o - m)
    o_ref[...] = e / jnp.sum(e, axis=0, keepdims=True)


def _mlp_t_entry(x0, x1, x2, x3, x4, w1t, b1, w2t, b2, w3t, b3, o):
    _mlp_t_body((x0, x1, x2, x3, x4), w1t, b1, w2t, b2, w3t, b3, o)


_NB = B // _BLK


def _x_spec(l):
    return pl.BlockSpec((E, _BLK), lambda i, _l=l: (0, _l * _NB + i))


_mlp_t = pl.pallas_call(
    _mlp_t_entry,
    grid=(_NB,),
    in_specs=[_x_spec(l) for l in range(L)] + [
        pl.BlockSpec((H, L * E), lambda i: (0, 0)),
        pl.BlockSpec((H, 1), lambda i: (0, 0)),
        pl.BlockSpec((H, H), lambda i: (0, 0)),
        pl.BlockSpec((H, 1), lambda i: (0, 0)),
        pl.BlockSpec((H, H), lambda i: (0, 0)),
        pl.BlockSpec((H, 1), lambda i: (0, 0)),
    ],
    out_specs=pl.BlockSpec((H, _BLK), lambda i: (0, i)),
    out_shape=jax.ShapeDtypeStruct((H, B), jnp.float32),
)


def kernel(inputs_BL, emb, W1, b1, W2, b2, W3, b3):
    idx = inputs_BL.astype(jnp.int32).T.reshape(-1)
    rows = jnp.take(emb.astype(jnp.bfloat16), idx, axis=0)
    xT = rows.T
    w3p = jnp.pad(W3, ((0, 0), (0, H - C)))
    bf = jnp.bfloat16
    oT = _mlp_t(
        xT, xT, xT, xT, xT,
        W1.T.astype(bf), b1.reshape(H, 1),
        W2.T.astype(bf), b2.reshape(H, 1),
        w3p.T.astype(bf), jnp.pad(b3, (0, H - C)).reshape(H, 1),
    )
    return oT[:C, :].T
